# Initial kernel scaffold; baseline (speedup 1.0000x reference)
import jax
import jax.numpy as jnp
from jax.experimental import pallas as pl


def kernel(char, lang, source_embedding, lang_embedding, fc_w, fc_b):
    raise NotImplementedError("write your pallas kernel here")


# speedup vs baseline: 3.2062x; 3.2062x over previous
import functools

import jax
import jax.numpy as jnp
from jax import lax
from jax.experimental import pallas as pl
from jax.experimental.pallas import tpu as pltpu
from jax.experimental.pallas import tpu_sc as plsc

_VOCAB = 256
_N_LANGS = 100
_D = 64
_B = 4096
_S = 50
_NTOK = _B * _S

_info = plsc.get_sparse_core_info()
_NC, _NS = _info.num_cores, _info.num_subcores
_NW = _NC * _NS
_TOK_PER_W = _NTOK // _NW
_CHUNK = 128
_NCHUNK = _TOK_PER_W // _CHUNK


def _tables_body(char_ref, lang_ref, se_ref, le_ref, w_ref, b_ref,
                 table_ref, idx_ref):
    se = se_ref[...]
    le = le_ref[...]
    w = w_ref[...]
    b = b_ref[...]
    sp = lax.dot_general(se, w[:, :_D], (((1,), (1,)), ((), ())),
                         preferred_element_type=jnp.float32)
    lp = lax.dot_general(le, w[:, _D:], (((1,), (1,)), ((), ())),
                         preferred_element_type=jnp.float32) + b
    table_ref[...] = sp[:, None, :] + lp[None, :, :]
    idx_ref[...] = char_ref[...] * _N_LANGS + lang_ref[...]


def _build_tables(char, lang2, se, le, w, b2):
    return pl.pallas_call(
        _tables_body,
        out_shape=(
            jax.ShapeDtypeStruct((_VOCAB, _N_LANGS, _D), jnp.float32),
            jax.ShapeDtypeStruct((_B, _S), jnp.int32),
        ),
    )(char, lang2, se, le, w, b2)


_mesh = plsc.VectorSubcoreMesh(core_axis_name="c", subcore_axis_name="s")


@functools.partial(
    pl.kernel,
    mesh=_mesh,
    compiler_params=pltpu.CompilerParams(use_tc_tiling_on_sc=False),
    out_type=jax.ShapeDtypeStruct((_NTOK, _D), jnp.float32),
    scratch_types=[
        pltpu.VMEM((_CHUNK,), jnp.int32),
        pltpu.VMEM((_CHUNK, _D), jnp.float32),
        pltpu.SemaphoreType.DMA,
    ],
)
def _sc_gather(table_hbm, idx_hbm, out_hbm, idx_v, rows_v, sem):
    wid = lax.axis_index("s") * _NC + lax.axis_index("c")
    base = wid * _TOK_PER_W

    def body(j, _):
        off = base + j * _CHUNK
        pltpu.sync_copy(idx_hbm.at[pl.ds(off, _CHUNK)], idx_v)
        pltpu.async_copy(table_hbm.at[idx_v], rows_v, sem).wait()
        pltpu.sync_copy(rows_v, out_hbm.at[pl.ds(off, _CHUNK)])
        return 0

    lax.fori_loop(0, _NCHUNK, body, 0)


def kernel(char, lang, source_embedding, lang_embedding, fc_w, fc_b):
    table3, idx = _build_tables(char, lang[:, None], source_embedding,
                                lang_embedding, fc_w, fc_b[None, :])
    table = table3.reshape(_VOCAB * _N_LANGS, _D)
    out = _sc_gather(table, idx.reshape(_NTOK))
    return out.reshape(_B, _S, _D)


# speedup vs baseline: 4.0430x; 1.2610x over previous
import functools

import jax
import jax.numpy as jnp
from jax import lax
from jax.experimental import pallas as pl
from jax.experimental.pallas import tpu as pltpu
from jax.experimental.pallas import tpu_sc as plsc

_VOCAB = 256
_N_LANGS = 100
_D = 64
_B = 4096
_S = 50
_NTOK = _B * _S

_info = plsc.get_sparse_core_info()
_NC, _NS = _info.num_cores, _info.num_subcores
_NW = _NC * _NS
_TOK_PER_W = _NTOK // _NW
_CHUNK = 128
_NCHUNK = _TOK_PER_W // _CHUNK


def _tables_body(char_ref, lang_ref, se_ref, le_ref, w_ref, b_ref,
                 table_ref, idx_ref):
    se = se_ref[...]
    le = le_ref[...]
    w = w_ref[...]
    b = b_ref[...]
    sp = lax.dot_general(se, w[:, :_D], (((1,), (1,)), ((), ())),
                         preferred_element_type=jnp.float32)
    lp = lax.dot_general(le, w[:, _D:], (((1,), (1,)), ((), ())),
                         preferred_element_type=jnp.float32) + b
    table_ref[...] = sp[:, None, :] + lp[None, :, :]
    idx_ref[...] = char_ref[...] * _N_LANGS + lang_ref[...]


def _build_tables(char, lang2, se, le, w, b2):
    return pl.pallas_call(
        _tables_body,
        out_shape=(
            jax.ShapeDtypeStruct((_VOCAB, _N_LANGS, _D), jnp.float32),
            jax.ShapeDtypeStruct((_B, _S), jnp.int32),
        ),
    )(char, lang2, se, le, w, b2)


_mesh = plsc.VectorSubcoreMesh(core_axis_name="c", subcore_axis_name="s")

_K = 5
_GROUP = _K * _CHUNK
_NGROUP = _TOK_PER_W // _GROUP
_NPAIR = _NGROUP // 2


@functools.partial(
    pl.kernel,
    mesh=_mesh,
    compiler_params=pltpu.CompilerParams(use_tc_tiling_on_sc=False),
    out_type=jax.ShapeDtypeStruct((_NTOK, _D), jnp.float32),
    scratch_types=[
        pltpu.VMEM((_NCHUNK, _CHUNK), jnp.int32),
        pltpu.VMEM((_GROUP, _D), jnp.float32),
        pltpu.VMEM((_GROUP, _D), jnp.float32),
        pltpu.SemaphoreType.DMA,
        pltpu.SemaphoreType.DMA,
        pltpu.SemaphoreType.DMA,
        pltpu.SemaphoreType.DMA,
    ],
)
def _sc_gather(table_hbm, idx_hbm, out_hbm, idx_v, rows_a, rows_b,
               gsem_a, gsem_b, ssem_a, ssem_b):
    wid = lax.axis_index("s") * _NC + lax.axis_index("c")
    base = wid * _TOK_PER_W

    pltpu.sync_copy(idx_hbm.at[pl.ds(wid * _NCHUNK, _NCHUNK)], idx_v)

    def fire_gathers(c0, rows, gsem):
        for b in range(_K):
            pltpu.async_copy(table_hbm.at[idx_v.at[c0 + b]],
                             rows.at[pl.ds(b * _CHUNK, _CHUNK)], gsem)

    def drain_gathers(c0, rows, gsem):
        for b in range(_K):
            pltpu.make_async_copy(table_hbm.at[idx_v.at[c0 + b]],
                                  rows.at[pl.ds(b * _CHUNK, _CHUNK)],
                                  gsem).wait()

    def fire_store(g, rows, ssem):
        pltpu.async_copy(rows, out_hbm.at[pl.ds(base + g * _GROUP, _GROUP)],
                         ssem)

    def wait_store(rows, ssem):
        pltpu.make_async_copy(rows, out_hbm.at[pl.ds(base, _GROUP)],
                              ssem).wait()

    fire_gathers(0, rows_a, gsem_a)

    def body(j, _):
        ga = 2 * j
        gb = 2 * j + 1

        @pl.when(j > 0)
        def _():
            wait_store(rows_b, ssem_b)
        fire_gathers(gb * _K, rows_b, gsem_b)
        drain_gathers(ga * _K, rows_a, gsem_a)
        fire_store(ga, rows_a, ssem_a)

        @pl.when(j < _NPAIR - 1)
        def _():
            wait_store(rows_a, ssem_a)
            fire_gathers((ga + 2) * _K, rows_a, gsem_a)
        drain_gathers(gb * _K, rows_b, gsem_b)
        fire_store(gb, rows_b, ssem_b)
        return 0

    lax.fori_loop(0, _NPAIR, body, 0)
    wait_store(rows_a, ssem_a)
    wait_store(rows_b, ssem_b)


def kernel(char, lang, source_embedding, lang_embedding, fc_w, fc_b):
    table3, idx = _build_tables(char, lang[:, None], source_embedding,
                                lang_embedding, fc_w, fc_b[None, :])
    table = table3.reshape(_VOCAB * _N_LANGS, _D)
    out = _sc_gather(table, idx.reshape(_NW * _NCHUNK, _CHUNK))
    return out.reshape(_B, _S, _D)
